# Initial kernel scaffold; baseline (speedup 1.0000x reference)
import jax
import jax.numpy as jnp
from jax.experimental import pallas as pl


def kernel(x, edge_index, Win_nb, Win_self, g_in, b_in, S1_nb, S1_self, S1_g, S1_b, Wd_nb, Wd_self, g_d, b_d, S2_nb, S2_self, S2_g, S2_b):
    raise NotImplementedError("write your pallas kernel here")


# speedup vs baseline: 10.0428x; 10.0428x over previous
import functools

import jax
import jax.numpy as jnp
from jax import lax
from jax.experimental import pallas as pl
from jax.experimental.pallas import tpu as pltpu
from jax.experimental.pallas import tpu_sc as plsc

NC = 2
NS = 16
NW = NC * NS
CHUNK = 128


@functools.partial(jax.jit, static_argnames=("n_pad", "c", "k_chunks"))
def _edge_scatter(y, src3, dst3, zeros, *, n_pad, c, k_chunks):
    rows_per_tile = n_pad // NS
    mesh = plsc.VectorSubcoreMesh(core_axis_name="c", subcore_axis_name="s")

    @functools.partial(
        pl.kernel,
        out_type=jax.ShapeDtypeStruct((NC, n_pad, c), jnp.float32),
        mesh=mesh,
        scratch_types=[
            pltpu.VMEM((k_chunks, CHUNK), jnp.int32),
            pltpu.VMEM((k_chunks, CHUNK), jnp.int32),
            pltpu.VMEM((CHUNK, c), jnp.float32),
            pltpu.VMEM_SHARED((n_pad, c), jnp.float32),
            pltpu.SemaphoreType.DMA,
        ],
        compiler_params=pltpu.CompilerParams(use_tc_tiling_on_sc=False),
    )
    def k(y_hbm, src_hbm, dst_hbm, z_hbm, out_hbm, src_v, dst_v, rows_v,
          agg_sh, sem):
        cid = lax.axis_index("c")
        sid = lax.axis_index("s")
        wid = cid * NS + sid
        pltpu.sync_copy(src_hbm.at[wid], src_v)
        pltpu.sync_copy(dst_hbm.at[wid], dst_v)
        r0 = sid * rows_per_tile
        pltpu.sync_copy(z_hbm.at[pl.ds(r0, rows_per_tile)],
                        agg_sh.at[pl.ds(r0, rows_per_tile)])
        plsc.subcore_barrier()

        def body(j, carry):
            pltpu.async_copy(y_hbm.at[src_v.at[j]], rows_v, sem).wait()
            pltpu.sync_copy(rows_v, agg_sh.at[dst_v.at[j]], add=True)
            return carry

        lax.fori_loop(0, k_chunks, body, 0)
        plsc.subcore_barrier()
        pltpu.sync_copy(agg_sh.at[pl.ds(r0, rows_per_tile)],
                        out_hbm.at[cid, pl.ds(r0, rows_per_tile)])

    return k(y, src3, dst3, zeros)


def _mm2(h, wnb, wself):
    n = h.shape[0]
    cout = wnb.shape[1]

    def body(h_ref, a_ref, b_ref, y_ref, s_ref):
        hh = h_ref[...]
        y_ref[...] = jnp.dot(hh, a_ref[...],
                             preferred_element_type=jnp.float32)
        s_ref[...] = jnp.dot(hh, b_ref[...],
                             preferred_element_type=jnp.float32)

    return pl.pallas_call(
        body,
        out_shape=(jax.ShapeDtypeStruct((n, cout), jnp.float32),
                   jax.ShapeDtypeStruct((n, cout), jnp.float32)),
    )(h, wnb, wself)


def _combine(parts, s, g, b, res):
    n, c = s.shape
    inv_n = 1.0 / n

    def body(*refs):
        if res is None:
            p_ref, s_ref, g_ref, b_ref, o_ref = refs
            r = None
        else:
            p_ref, s_ref, g_ref, b_ref, r_ref, o_ref = refs
            r = r_ref[...]
        pre = p_ref[0, :n, :] + p_ref[1, :n, :] + s_ref[...]
        m = jnp.sum(pre, axis=0, keepdims=True) * inv_n
        d = pre - m
        v = jnp.sum(d * d, axis=0, keepdims=True) * inv_n
        hn = d * lax.rsqrt(v + 1e-3) * g_ref[...] + b_ref[...]
        if r is not None:
            hn = hn + r
        o_ref[...] = jnp.maximum(hn, 0.0)

    args = [parts, s, g.reshape(1, c), b.reshape(1, c)]
    if res is not None:
        args.append(res)
    return pl.pallas_call(
        body,
        out_shape=jax.ShapeDtypeStruct((n, c), jnp.float32),
    )(*args)


def _layer(h, wnb, wself, g, b, src3, dst3, zeros, n_pad, k_chunks, res=None):
    y, s = _mm2(h, wnb, wself)
    parts = _edge_scatter(y, src3, dst3, zeros,
                          n_pad=n_pad, c=y.shape[1], k_chunks=k_chunks)
    return _combine(parts, s, g, b, res)


def kernel(x, edge_index, Win_nb, Win_self, g_in, b_in, S1_nb, S1_self,
           S1_g, S1_b, Wd_nb, Wd_self, g_d, b_d, S2_nb, S2_self, S2_g, S2_b):
    n = x.shape[0]
    e = edge_index.shape[1]
    n_pad = -(-(n + 1) // (NS * 8)) * (NS * 8)
    k_chunks = -(-e // (NW * CHUNK))
    e_pad = NW * k_chunks * CHUNK

    src = edge_index[0].astype(jnp.int32)
    dst = edge_index[1].astype(jnp.int32)
    src3 = jnp.concatenate(
        [src, jnp.zeros((e_pad - e,), jnp.int32)]).reshape(NW, k_chunks, CHUNK)
    dst3 = jnp.concatenate(
        [dst, jnp.full((e_pad - e,), n, jnp.int32)]).reshape(NW, k_chunks, CHUNK)

    z16 = jnp.zeros((n_pad, 16), jnp.float32)
    z32 = jnp.zeros((n_pad, 32), jnp.float32)

    def mp(h, wnb, wself, g, b, res=None):
        z = z16 if wnb.shape[1] == 16 else z32
        return _layer(h, wnb, wself, g, b, src3, dst3, z, n_pad, k_chunks,
                      res=res)

    h = mp(x, Win_nb, Win_self, g_in, b_in)
    for i in range(2):
        out = mp(h, S1_nb[i, 0], S1_self[i, 0], S1_g[i, 0], S1_b[i, 0])
        h = mp(out, S1_nb[i, 1], S1_self[i, 1], S1_g[i, 1], S1_b[i, 1],
               res=h)
    h = mp(h, Wd_nb, Wd_self, g_d, b_d)
    for i in range(2):
        out = mp(h, S2_nb[i, 0], S2_self[i, 0], S2_g[i, 0], S2_b[i, 0])
        h = mp(out, S2_nb[i, 1], S2_self[i, 1], S2_g[i, 1], S2_b[i, 1],
               res=h)
    return h


# speedup vs baseline: 11.4640x; 1.1415x over previous
import functools

import jax
import jax.numpy as jnp
from jax import lax
from jax.experimental import pallas as pl
from jax.experimental.pallas import tpu as pltpu
from jax.experimental.pallas import tpu_sc as plsc

NC = 2
NS = 16
NW = NC * NS
CHUNK = 128
NBUF = 8


@functools.partial(jax.jit, static_argnames=("n_pad", "c", "k_chunks"))
def _edge_scatter(y, src3, dst3, zeros, *, n_pad, c, k_chunks):
    rows_per_tile = n_pad // NS
    mesh = plsc.VectorSubcoreMesh(core_axis_name="c", subcore_axis_name="s")

    @functools.partial(
        pl.kernel,
        out_type=jax.ShapeDtypeStruct((NC, n_pad, c), jnp.float32),
        mesh=mesh,
        scratch_types=[
            pltpu.VMEM((k_chunks, CHUNK), jnp.int32),
            pltpu.VMEM((k_chunks, CHUNK), jnp.int32),
            pltpu.VMEM((NBUF, CHUNK, c), jnp.float32),
            pltpu.VMEM_SHARED((n_pad, c), jnp.float32),
            pltpu.SemaphoreType.DMA((NBUF,)),
        ],
        compiler_params=pltpu.CompilerParams(use_tc_tiling_on_sc=False),
    )
    def k(y_hbm, src_hbm, dst_hbm, z_hbm, out_hbm, src_v, dst_v, rows_v,
          agg_sh, sem):
        cid = lax.axis_index("c")
        sid = lax.axis_index("s")
        wid = cid * NS + sid
        pltpu.sync_copy(src_hbm.at[wid], src_v)
        pltpu.sync_copy(dst_hbm.at[wid], dst_v)
        r0 = sid * rows_per_tile
        pltpu.sync_copy(z_hbm.at[pl.ds(r0, rows_per_tile)],
                        agg_sh.at[pl.ds(r0, rows_per_tile)])
        plsc.subcore_barrier()

        for b in range(NBUF):
            pltpu.async_copy(y_hbm.at[src_v.at[b]], rows_v.at[b], sem.at[b])

        def group(g, carry):
            j0 = g * NBUF
            for b in range(NBUF):
                j = j0 + b
                pltpu.make_async_copy(y_hbm.at[src_v.at[j]], rows_v.at[b],
                                      sem.at[b]).wait()
                pltpu.sync_copy(rows_v.at[b], agg_sh.at[dst_v.at[j]],
                                add=True)
                nj = j + NBUF

                @pl.when(nj < k_chunks)
                def _():
                    pltpu.async_copy(y_hbm.at[src_v.at[nj]], rows_v.at[b],
                                     sem.at[b])
            return carry

        lax.fori_loop(0, k_chunks // NBUF, group, 0)
        plsc.subcore_barrier()
        pltpu.sync_copy(agg_sh.at[pl.ds(r0, rows_per_tile)],
                        out_hbm.at[cid, pl.ds(r0, rows_per_tile)])

    return k(y, src3, dst3, zeros)


def _mm2(h, wnb, wself):
    n = h.shape[0]
    cout = wnb.shape[1]

    def body(h_ref, a_ref, b_ref, y_ref, s_ref):
        hh = h_ref[...]
        y_ref[...] = jnp.dot(hh, a_ref[...],
                             preferred_element_type=jnp.float32)
        s_ref[...] = jnp.dot(hh, b_ref[...],
                             preferred_element_type=jnp.float32)

    return pl.pallas_call(
        body,
        out_shape=(jax.ShapeDtypeStruct((n, cout), jnp.float32),
                   jax.ShapeDtypeStruct((n, cout), jnp.float32)),
    )(h, wnb, wself)


def _combine(parts, s, g, b, res):
    n, c = s.shape
    inv_n = 1.0 / n

    def body(*refs):
        if res is None:
            p_ref, s_ref, g_ref, b_ref, o_ref = refs
            r = None
        else:
            p_ref, s_ref, g_ref, b_ref, r_ref, o_ref = refs
            r = r_ref[...]
        pre = p_ref[0, :n, :] + p_ref[1, :n, :] + s_ref[...]
        m = jnp.sum(pre, axis=0, keepdims=True) * inv_n
        d = pre - m
        v = jnp.sum(d * d, axis=0, keepdims=True) * inv_n
        hn = d * lax.rsqrt(v + 1e-3) * g_ref[...] + b_ref[...]
        if r is not None:
            hn = hn + r
        o_ref[...] = jnp.maximum(hn, 0.0)

    args = [parts, s, g.reshape(1, c), b.reshape(1, c)]
    if res is not None:
        args.append(res)
    return pl.pallas_call(
        body,
        out_shape=jax.ShapeDtypeStruct((n, c), jnp.float32),
    )(*args)


def _layer(h, wnb, wself, g, b, src3, dst3, zeros, n_pad, k_chunks, res=None):
    y, s = _mm2(h, wnb, wself)
    parts = _edge_scatter(y, src3, dst3, zeros,
                          n_pad=n_pad, c=y.shape[1], k_chunks=k_chunks)
    return _combine(parts, s, g, b, res)


def kernel(x, edge_index, Win_nb, Win_self, g_in, b_in, S1_nb, S1_self,
           S1_g, S1_b, Wd_nb, Wd_self, g_d, b_d, S2_nb, S2_self, S2_g, S2_b):
    n = x.shape[0]
    e = edge_index.shape[1]
    n_pad = -(-(n + 1) // (NS * 8)) * (NS * 8)
    k_chunks = -(-e // (NW * CHUNK * NBUF)) * NBUF
    e_pad = NW * k_chunks * CHUNK

    src = edge_index[0].astype(jnp.int32)
    dst = edge_index[1].astype(jnp.int32)
    src3 = jnp.concatenate(
        [src, jnp.zeros((e_pad - e,), jnp.int32)]).reshape(NW, k_chunks, CHUNK)
    dst3 = jnp.concatenate(
        [dst, jnp.full((e_pad - e,), n, jnp.int32)]).reshape(NW, k_chunks, CHUNK)

    z16 = jnp.zeros((n_pad, 16), jnp.float32)
    z32 = jnp.zeros((n_pad, 32), jnp.float32)

    def mp(h, wnb, wself, g, b, res=None):
        z = z16 if wnb.shape[1] == 16 else z32
        return _layer(h, wnb, wself, g, b, src3, dst3, z, n_pad, k_chunks,
                      res=res)

    h = mp(x, Win_nb, Win_self, g_in, b_in)
    for i in range(2):
        out = mp(h, S1_nb[i, 0], S1_self[i, 0], S1_g[i, 0], S1_b[i, 0])
        h = mp(out, S1_nb[i, 1], S1_self[i, 1], S1_g[i, 1], S1_b[i, 1],
               res=h)
    h = mp(h, Wd_nb, Wd_self, g_d, b_d)
    for i in range(2):
        out = mp(h, S2_nb[i, 0], S2_self[i, 0], S2_g[i, 0], S2_b[i, 0])
        h = mp(out, S2_nb[i, 1], S2_self[i, 1], S2_g[i, 1], S2_b[i, 1],
               res=h)
    return h
